# baseline (speedup 1.0000x reference)
import functools

import jax
import jax.numpy as jnp
import numpy as np
from jax import lax
from jax.experimental import pallas as pl
from jax.experimental.pallas import tpu as pltpu
from jax.experimental.pallas import tpu_sc as plsc

B_TOTAL = 4 * 256 * 256
D = 512
V = 64
NC = 2
NS = 16
NW = NC * NS
T = 8
NTPD = 256 // T
NDIAG = 4 * NTPD // NW
NUNIT = NDIAG + 4 * (NTPD * (NTPD - 1) // 2) // NW
UROWS = T * T
WROWS = NUNIT * UROWS
SPAD = 72


def _build_schedule():
    diags = [(b, i) for b in range(4) for i in range(NTPD)]
    pairs = [(b, i, j)
             for b in range(4)
             for i in range(NTPD)
             for j in range(i + 1, NTPD)]
    perm = np.zeros(NW * WROWS, np.int32)
    sa = np.zeros(NW * SPAD, np.int32)
    sb = np.zeros(NW * SPAD, np.int32)
    rr, jj = np.meshgrid(np.arange(T), np.arange(T), indexing="ij")
    for w in range(NW):
        units = [(b, i, i) for (b, i) in diags[w * NDIAG:(w + 1) * NDIAG]]
        units += pairs[w * (NUNIT - NDIAG):(w + 1) * (NUNIT - NDIAG)]
        for u, (b, bi, bj) in enumerate(units):
            base = (w * NUNIT + u) * UROWS
            pos = b * 65536 + (bi * T + rr) * 256 + (bj * T + jj)
            perm[base:base + UROWS] = pos.reshape(-1)
            sa[w * SPAD + u] = (b * 256 + bi * T) * 256 + bj * T
            sb[w * SPAD + u] = (b * 256 + bj * T) * 256 + bi * T
    return perm, sa, sb


_PERM_NP, _SA_NP, _SB_NP = _build_schedule()


def _sc_lookup(a_perm, b_perm, table_rep, sa_arr, sb_arr):
    mesh = plsc.VectorSubcoreMesh(core_axis_name="c", subcore_axis_name="s")

    @functools.partial(
        pl.kernel,
        mesh=mesh,
        compiler_params=pltpu.CompilerParams(use_tc_tiling_on_sc=False),
        out_type=jax.ShapeDtypeStruct((B_TOTAL, D), jnp.float32),
        scratch_types=[
            pltpu.VMEM((WROWS + 16,), jnp.int32),
            pltpu.VMEM((WROWS,), jnp.int32),
            pltpu.VMEM((SPAD + 16,), jnp.int32),
            pltpu.VMEM((SPAD + 16,), jnp.int32),
            pltpu.VMEM((2, UROWS, D), jnp.float32),
            pltpu.SemaphoreType.DMA,
            pltpu.SemaphoreType.DMA,
            pltpu.SemaphoreType.DMA,
        ],
    )
    def body(a_hbm, b_hbm, table_hbm, sa_hbm, sb_hbm, out_hbm,
             idx_v, add_v, sa_v, sb_v, rows_v, gsem0, gsem1, osem):
        wid = lax.axis_index("s") * NC + lax.axis_index("c")
        base = wid * WROWS
        toff = wid * V

        pltpu.sync_copy(a_hbm.at[pl.ds(base, WROWS)],
                        idx_v.at[pl.ds(0, WROWS)])
        pltpu.sync_copy(b_hbm.at[pl.ds(base, WROWS)], add_v)
        pltpu.sync_copy(sa_hbm.at[pl.ds(wid * SPAD, SPAD)],
                        sa_v.at[pl.ds(0, SPAD)])
        pltpu.sync_copy(sb_hbm.at[pl.ds(wid * SPAD, SPAD)],
                        sb_v.at[pl.ds(0, SPAD)])

        def add_chunk(i, carry):
            sl = pl.ds(i * 16, 16)
            idx_v[sl] = idx_v[sl] + add_v[sl] + toff
            return carry

        lax.fori_loop(0, WROWS // 16, add_chunk, 0)

        def start_gather(u, slot, sem):
            pltpu.async_copy(
                table_hbm.at[idx_v.at[pl.ds(u * UROWS, UROWS)]],
                rows_v.at[slot], sem)

        def wait_gather(slot, sem):
            pltpu.make_async_copy(
                table_hbm.at[pl.ds(0, UROWS)], rows_v.at[slot], sem).wait()

        def write_primary(u, slot):
            sa = sa_v[pl.ds(u, 16)][0]
            for r in range(T):
                pltpu.async_copy(
                    rows_v.at[slot].at[pl.ds(r * T, T)],
                    out_hbm.at[pl.ds(sa + r * 256, T)], osem)

        def write_mirror(u, slot):
            sb = sb_v[pl.ds(u, 16)][0]
            for j in range(T):
                for r in range(T):
                    pltpu.async_copy(
                        rows_v.at[slot].at[pl.ds(r * T + j, 1)],
                        out_hbm.at[pl.ds(sb + j * 256 + r, 1)], osem)

        def wait_write(slot):
            pltpu.make_async_copy(
                rows_v.at[slot], out_hbm.at[pl.ds(0, UROWS)], osem).wait()

        start_gather(0, 0, gsem0)

        for u in range(NDIAG):
            slot = u % 2
            wait_gather(slot, (gsem0, gsem1)[slot])
            write_primary(u, slot)
            wait_write(slot)
            start_gather(u + 1, (u + 1) % 2, (gsem0, gsem1)[(u + 1) % 2])

        start_gather(NDIAG + 1, 1, gsem1)

        def unit_pair(t, carry):
            ua = NDIAG + 2 * t
            ub = ua + 1
            last = (NUNIT - NDIAG) // 2 - 1
            wait_gather(0, gsem0)
            write_primary(ua, 0)
            write_mirror(ua, 0)
            wait_gather(1, gsem1)
            wait_write(0)
            wait_write(0)

            @pl.when(t < last)
            def _():
                start_gather(ua + 2, 0, gsem0)

            write_primary(ub, 1)
            write_mirror(ub, 1)
            wait_write(1)
            wait_write(1)

            @pl.when(t < last)
            def _():
                start_gather(ub + 2, 1, gsem1)

            return carry

        lax.fori_loop(0, (NUNIT - NDIAG) // 2, unit_pair, 0)

    return body(a_perm, b_perm, table_rep, sa_arr, sb_arr)


def kernel(graphs, spec_type, normal_type):
    table = jnp.concatenate((spec_type, normal_type), axis=0)
    table_rep = jnp.tile(table, (NW, 1))
    perm = jnp.asarray(_PERM_NP)
    g_flat = graphs.reshape(B_TOTAL)
    gt_flat = jnp.transpose(graphs, (0, 2, 1)).reshape(B_TOTAL)
    a_perm = g_flat[perm]
    b_perm = gt_flat[perm]
    out = _sc_lookup(a_perm, b_perm, table_rep,
                     jnp.asarray(_SA_NP), jnp.asarray(_SB_NP))
    return out.reshape(4, 256, 256, D)


# speedup vs baseline: 2.0566x; 2.0566x over previous
import functools

import jax
import jax.numpy as jnp
from jax import lax
from jax.experimental import pallas as pl
from jax.experimental.pallas import tpu as pltpu
from jax.experimental.pallas import tpu_sc as plsc

B_TOTAL = 4 * 256 * 256
D = 512
V = 64
NC = 2
NS = 16
NW = NC * NS
BPW = B_TOTAL // NW
K = 64
NCHUNK = BPW // K
NPAIR = NCHUNK // 2


def _sc_lookup(g_flat, gt_flat, table_rep):
    mesh = plsc.VectorSubcoreMesh(core_axis_name="c", subcore_axis_name="s")

    @functools.partial(
        pl.kernel,
        mesh=mesh,
        out_type=jax.ShapeDtypeStruct((B_TOTAL, D), jnp.float32),
        scratch_types=[
            pltpu.VMEM((BPW,), jnp.int32),
            pltpu.VMEM((BPW,), jnp.int32),
            pltpu.VMEM((2, K, D), jnp.float32),
            pltpu.SemaphoreType.DMA,
            pltpu.SemaphoreType.DMA,
            pltpu.SemaphoreType.DMA,
            pltpu.SemaphoreType.DMA,
        ],
    )
    def body(g_hbm, gt_hbm, table_hbm, out_hbm, idx_v, add_v, rows_v,
             gsem0, gsem1, osem0, osem1):
        wid = lax.axis_index("s") * NC + lax.axis_index("c")
        base = wid * BPW
        toff = wid * V

        pltpu.sync_copy(g_hbm.at[pl.ds(base, BPW)], idx_v)
        pltpu.sync_copy(gt_hbm.at[pl.ds(base, BPW)], add_v)

        def add_chunk(i, carry):
            sl = pl.ds(i * 16, 16)
            idx_v[sl] = idx_v[sl] + add_v[sl] + toff
            return carry

        lax.fori_loop(0, BPW // 16, add_chunk, 0)

        def start_gather(c, slot, sem):
            pltpu.async_copy(
                table_hbm.at[idx_v.at[pl.ds(c * K, K)]], rows_v.at[slot], sem)

        def wait_gather(slot, sem):
            pltpu.make_async_copy(
                table_hbm.at[pl.ds(0, K)], rows_v.at[slot], sem).wait()

        def start_out(c, slot, sem):
            pltpu.async_copy(
                rows_v.at[slot], out_hbm.at[pl.ds(base + c * K, K)], sem)

        def wait_out(slot, sem):
            pltpu.make_async_copy(
                rows_v.at[slot], out_hbm.at[pl.ds(base, K)], sem).wait()

        start_gather(0, 0, gsem0)

        def pair(p, carry):
            a = 2 * p
            b = a + 1
            wait_gather(0, gsem0)

            @pl.when(p > 0)
            def _():
                wait_out(1, osem1)

            start_gather(b, 1, gsem1)
            start_out(a, 0, osem0)
            wait_gather(1, gsem1)
            wait_out(0, osem0)

            @pl.when(p < NPAIR - 1)
            def _():
                start_gather(a + 2, 0, gsem0)

            start_out(b, 1, osem1)
            return carry

        lax.fori_loop(0, NPAIR, pair, 0)
        wait_out(1, osem1)

    return body(g_flat, gt_flat, table_rep)


def kernel(graphs, spec_type, normal_type):
    table = jnp.concatenate((spec_type, normal_type), axis=0)
    table_rep = jnp.tile(table, (NW, 1))
    g_flat = graphs.reshape(B_TOTAL)
    gt_flat = jnp.transpose(graphs, (0, 2, 1)).reshape(B_TOTAL)
    out = _sc_lookup(g_flat, gt_flat, table_rep)
    return out.reshape(4, 256, 256, D)
